# Initial kernel scaffold; baseline (speedup 1.0000x reference)
import jax
import jax.numpy as jnp
from jax.experimental import pallas as pl


def kernel(tl_heat_0, br_heat_0, tl_regr_0, br_regr_0, tl_c_regr_0, br_c_regr_0, tl_heat_1, br_heat_1, tl_regr_1, br_regr_1, tl_c_regr_1, br_c_regr_1, tl_heat_2, br_heat_2, tl_regr_2, br_regr_2, tl_c_regr_2, br_c_regr_2, tl_heat_3, br_heat_3, tl_regr_3, br_regr_3, tl_c_regr_3, br_c_regr_3, tl_heat_4, br_heat_4, tl_regr_4, br_regr_4, tl_c_regr_4, br_c_regr_4):
    raise NotImplementedError("write your pallas kernel here")


# speedup vs baseline: 1.0878x; 1.0878x over previous
import functools

import jax
import jax.numpy as jnp
import numpy as np
from jax import lax
from jax.experimental import pallas as pl
from jax.experimental.pallas import tpu as pltpu

_B = 4
_CAT = 80
_K = 100
_NUM_DETS = 1000
_DIST_T = 0.2
_NL = 5
_SIZES = [(128, 128), (64, 64), (32, 32), (16, 16), (8, 8)]
_LAYERS_RANGE = [[0.0, 96.0, 0.0, 96.0] for _ in range(_NL)]
_NEG = -jnp.inf


def _nms_body(x_ref, o_ref):
    x = x_ref[0]
    cat, h, w = x.shape
    neg_col = jnp.full((cat, h, 1), _NEG, dtype=x.dtype)
    xl = jnp.concatenate([x[:, :, 1:], neg_col], axis=2)
    xr = jnp.concatenate([neg_col, x[:, :, :-1]], axis=2)
    mw = jnp.maximum(jnp.maximum(xl, xr), x)
    neg_row = jnp.full((cat, 1, w), _NEG, dtype=x.dtype)
    mu = jnp.concatenate([mw[:, 1:, :], neg_row], axis=1)
    md = jnp.concatenate([neg_row, mw[:, :-1, :]], axis=1)
    hmax = jnp.maximum(jnp.maximum(mu, md), mw)
    o_ref[0] = jnp.where(hmax == x, x, 0.0)


def _nms_pallas(heat):
    n, cat, h, w = heat.shape
    return pl.pallas_call(
        _nms_body,
        grid=(n,),
        in_specs=[pl.BlockSpec((1, cat, h, w), lambda i: (i, 0, 0, 0))],
        out_specs=pl.BlockSpec((1, cat, h, w), lambda i: (i, 0, 0, 0)),
        out_shape=jax.ShapeDtypeStruct((n, cat, h, w), heat.dtype),
    )(heat)


def _topk_stage(scores, k):
    b, c, h, w = scores.shape
    topk_scores, topk_inds = lax.top_k(scores.reshape(b, -1), k)
    topk_clses = topk_inds // (h * w)
    topk_inds = topk_inds % (h * w)
    topk_ys = (topk_inds // w).astype(jnp.float32)
    topk_xs = (topk_inds % w).astype(jnp.float32)
    return topk_scores, topk_inds, topk_clses, topk_ys, topk_xs


def _gather_rows(feat, ind):
    b, n, d = feat.shape
    idx = jnp.broadcast_to(ind[:, :, None], (b, ind.shape[1], d))
    return jnp.take_along_axis(feat, idx, axis=1)


def _gather_chw(feat, ind):
    b, c, h, w = feat.shape
    f = jnp.transpose(feat, (0, 2, 3, 1)).reshape(b, h * w, c)
    return _gather_rows(f, ind)


def _decode_all(tl_heats, br_heats, tl_regrs, br_regrs, tl_c_regrs, br_c_regrs):
    batch, cat, h0, w0 = tl_heats[0].shape
    dets = []
    for i in range(_NL):
        b, c, h, w = tl_heats[i].shape
        hs = h0 / h
        ws = w0 / w
        both = jnp.concatenate([tl_heats[i], br_heats[i]], axis=0)
        both = _nms_pallas(both)
        tl_heat, br_heat = both[:b], both[b:]
        tl_s, tl_inds, tl_cls, tl_ys, tl_xs = _topk_stage(tl_heat, _K)
        br_s, br_inds, br_cls, br_ys, br_xs = _topk_stage(br_heat, _K)
        tl_ys = jnp.broadcast_to(tl_ys[:, :, None], (b, _K, _K))
        tl_xs = jnp.broadcast_to(tl_xs[:, :, None], (b, _K, _K))
        br_ys = jnp.broadcast_to(br_ys[:, None, :], (b, _K, _K))
        br_xs = jnp.broadcast_to(br_xs[:, None, :], (b, _K, _K))
        tl_r = _gather_chw(tl_regrs[i], tl_inds).reshape(b, _K, 1, 2)
        br_r = _gather_chw(br_regrs[i], br_inds).reshape(b, 1, _K, 2)
        tl_cr = _gather_chw(tl_c_regrs[i], tl_inds).reshape(b, _K, 1, 2)
        br_cr = _gather_chw(br_c_regrs[i], br_inds).reshape(b, 1, _K, 2)
        tl_xs = tl_xs + tl_r[..., 0]
        tl_ys = tl_ys + tl_r[..., 1]
        br_xs = br_xs + br_r[..., 0]
        br_ys = br_ys + br_r[..., 1]
        bboxes = jnp.stack((tl_xs, tl_ys, br_xs, br_ys), axis=3)
        bw = br_xs - tl_xs
        bh = br_ys - tl_ys
        distsx = jnp.abs(1.0 - _SIZES[-1][1] * (tl_cr[..., 0] + br_cr[..., 0]) / bw)
        distsy = jnp.abs(1.0 - _SIZES[-1][0] * (tl_cr[..., 1] + br_cr[..., 1]) / bh)
        dd = jnp.abs(br_cr - tl_cr)
        dists = dd[..., 1] + dd[..., 0]
        tl_se = jnp.broadcast_to(tl_s[:, :, None], (b, _K, _K))
        br_se = jnp.broadcast_to(br_s[:, None, :], (b, _K, _K))
        scores = (tl_se + br_se) / 2.0
        tl_ce = jnp.broadcast_to(tl_cls[:, :, None], (b, _K, _K))
        br_ce = jnp.broadcast_to(br_cls[:, None, :], (b, _K, _K))
        cls_inds = tl_ce != br_ce
        lr = _LAYERS_RANGE[i]
        wrange_ind = (bw < 0.8 * lr[2]) | (bw > 1.3 * lr[3])
        hrange_ind = (bh < 0.8 * lr[0]) | (bh > 1.3 * lr[1])
        scores = jnp.where(wrange_ind, -1.0, scores)
        scores = jnp.where(hrange_ind, -1.0, scores)
        dist_inds = (distsx > _DIST_T) | (distsy > _DIST_T) | (dists > 0.25)
        width_inds = br_xs < tl_xs
        height_inds = br_ys < tl_ys
        scores = jnp.where(cls_inds, -1.0, scores)
        scores = jnp.where(dist_inds, -1.0, scores)
        scores = jnp.where(width_inds, -1.0, scores)
        scores = jnp.where(height_inds, -1.0, scores)
        scores = scores.reshape(b, -1)
        scores, inds = lax.top_k(scores, min(_NUM_DETS, scores.shape[1]))
        scores = scores[:, :, None]
        bb = _gather_rows(bboxes.reshape(b, -1, 4), inds)
        clses = _gather_rows(tl_ce.reshape(b, -1, 1), inds).astype(jnp.float32)
        tl_sg = _gather_rows(tl_se.reshape(b, -1, 1), inds).astype(jnp.float32)
        br_sg = _gather_rows(br_se.reshape(b, -1, 1), inds).astype(jnp.float32)
        bb = bb * jnp.array([ws, hs, ws, hs], dtype=jnp.float32)
        dets.append(jnp.concatenate([bb, scores, tl_sg, br_sg, clses], axis=2))
    detections = jnp.concatenate(dets, axis=1)
    top_scores, top_inds = lax.top_k(detections[:, :, 4], 5 * _NUM_DETS)
    detections = _gather_rows(detections, top_inds)
    return detections


def kernel(tl_heat_0, br_heat_0, tl_regr_0, br_regr_0, tl_c_regr_0, br_c_regr_0,
           tl_heat_1, br_heat_1, tl_regr_1, br_regr_1, tl_c_regr_1, br_c_regr_1,
           tl_heat_2, br_heat_2, tl_regr_2, br_regr_2, tl_c_regr_2, br_c_regr_2,
           tl_heat_3, br_heat_3, tl_regr_3, br_regr_3, tl_c_regr_3, br_c_regr_3,
           tl_heat_4, br_heat_4, tl_regr_4, br_regr_4, tl_c_regr_4, br_c_regr_4):
    tl_heats = [tl_heat_0, tl_heat_1, tl_heat_2, tl_heat_3, tl_heat_4]
    br_heats = [br_heat_0, br_heat_1, br_heat_2, br_heat_3, br_heat_4]
    tl_regrs = [tl_regr_0, tl_regr_1, tl_regr_2, tl_regr_3, tl_regr_4]
    br_regrs = [br_regr_0, br_regr_1, br_regr_2, br_regr_3, br_regr_4]
    tl_c_regrs = [tl_c_regr_0, tl_c_regr_1, tl_c_regr_2, tl_c_regr_3, tl_c_regr_4]
    br_c_regrs = [br_c_regr_0, br_c_regr_1, br_c_regr_2, br_c_regr_3, br_c_regr_4]
    return _decode_all(tl_heats, br_heats, tl_regrs, br_regrs, tl_c_regrs, br_c_regrs)


# speedup vs baseline: 1.0879x; 1.0001x over previous
import jax
import jax.numpy as jnp
import numpy as np
from jax import lax
from jax.experimental import pallas as pl

_B = 4
_CAT = 80
_K = 100
_NUM_DETS = 1000
_DIST_T = 0.2
_NL = 5
_SIZES = [(128, 128), (64, 64), (32, 32), (16, 16), (8, 8)]
_LAYERS_RANGE = [[0.0, 96.0, 0.0, 96.0] for _ in range(_NL)]
_NEG = -jnp.inf


def _nms_body(x_ref, o_ref):
    x = x_ref[0]
    cat, h, w = x.shape
    neg_col = jnp.full((cat, h, 1), _NEG, dtype=x.dtype)
    xl = jnp.concatenate([x[:, :, 1:], neg_col], axis=2)
    xr = jnp.concatenate([neg_col, x[:, :, :-1]], axis=2)
    mw = jnp.maximum(jnp.maximum(xl, xr), x)
    neg_row = jnp.full((cat, 1, w), _NEG, dtype=x.dtype)
    mu = jnp.concatenate([mw[:, 1:, :], neg_row], axis=1)
    md = jnp.concatenate([neg_row, mw[:, :-1, :]], axis=1)
    hmax = jnp.maximum(jnp.maximum(mu, md), mw)
    o_ref[0] = jnp.where(hmax == x, x, 0.0)


def _nms_pallas(heat):
    n, cat, h, w = heat.shape
    return pl.pallas_call(
        _nms_body,
        grid=(n,),
        in_specs=[pl.BlockSpec((1, cat, h, w), lambda i: (i, 0, 0, 0))],
        out_specs=pl.BlockSpec((1, cat, h, w), lambda i: (i, 0, 0, 0)),
        out_shape=jax.ShapeDtypeStruct((n, cat, h, w), heat.dtype),
    )(heat)


def _topk_stage(scores, k):
    b, c, h, w = scores.shape
    topk_scores, topk_inds = lax.top_k(scores.reshape(b, -1), k)
    topk_clses = topk_inds // (h * w)
    topk_inds = topk_inds % (h * w)
    topk_ys = (topk_inds // w).astype(jnp.float32)
    topk_xs = (topk_inds % w).astype(jnp.float32)
    return topk_scores, topk_inds, topk_clses, topk_ys, topk_xs


def _gather_rows(feat, ind):
    b, n, d = feat.shape
    idx = jnp.broadcast_to(ind[:, :, None], (b, ind.shape[1], d))
    return jnp.take_along_axis(feat, idx, axis=1)


def _gather_chw(feat, ind):
    b, c, h, w = feat.shape
    f = jnp.transpose(feat, (0, 2, 3, 1)).reshape(b, h * w, c)
    return _gather_rows(f, ind)


def _decode_all(tl_heats, br_heats, tl_regrs, br_regrs, tl_c_regrs, br_c_regrs):
    batch, cat, h0, w0 = tl_heats[0].shape
    dets = []
    for i in range(_NL):
        b, c, h, w = tl_heats[i].shape
        hs = h0 / h
        ws = w0 / w
        both = jnp.concatenate([tl_heats[i], br_heats[i]], axis=0)
        both = _nms_pallas(both)
        tl_heat, br_heat = both[:b], both[b:]
        tl_s, tl_inds, tl_cls, tl_ys, tl_xs = _topk_stage(tl_heat, _K)
        br_s, br_inds, br_cls, br_ys, br_xs = _topk_stage(br_heat, _K)
        tl_ys = jnp.broadcast_to(tl_ys[:, :, None], (b, _K, _K))
        tl_xs = jnp.broadcast_to(tl_xs[:, :, None], (b, _K, _K))
        br_ys = jnp.broadcast_to(br_ys[:, None, :], (b, _K, _K))
        br_xs = jnp.broadcast_to(br_xs[:, None, :], (b, _K, _K))
        tl_r = _gather_chw(tl_regrs[i], tl_inds).reshape(b, _K, 1, 2)
        br_r = _gather_chw(br_regrs[i], br_inds).reshape(b, 1, _K, 2)
        tl_cr = _gather_chw(tl_c_regrs[i], tl_inds).reshape(b, _K, 1, 2)
        br_cr = _gather_chw(br_c_regrs[i], br_inds).reshape(b, 1, _K, 2)
        tl_xs = tl_xs + tl_r[..., 0]
        tl_ys = tl_ys + tl_r[..., 1]
        br_xs = br_xs + br_r[..., 0]
        br_ys = br_ys + br_r[..., 1]
        bboxes = jnp.stack((tl_xs, tl_ys, br_xs, br_ys), axis=3)
        bw = br_xs - tl_xs
        bh = br_ys - tl_ys
        distsx = jnp.abs(1.0 - _SIZES[-1][1] * (tl_cr[..., 0] + br_cr[..., 0]) / bw)
        distsy = jnp.abs(1.0 - _SIZES[-1][0] * (tl_cr[..., 1] + br_cr[..., 1]) / bh)
        dd = jnp.abs(br_cr - tl_cr)
        dists = dd[..., 1] + dd[..., 0]
        tl_se = jnp.broadcast_to(tl_s[:, :, None], (b, _K, _K))
        br_se = jnp.broadcast_to(br_s[:, None, :], (b, _K, _K))
        scores = (tl_se + br_se) / 2.0
        tl_ce = jnp.broadcast_to(tl_cls[:, :, None], (b, _K, _K))
        br_ce = jnp.broadcast_to(br_cls[:, None, :], (b, _K, _K))
        cls_inds = tl_ce != br_ce
        lr = _LAYERS_RANGE[i]
        wrange_ind = (bw < 0.8 * lr[2]) | (bw > 1.3 * lr[3])
        hrange_ind = (bh < 0.8 * lr[0]) | (bh > 1.3 * lr[1])
        scores = jnp.where(wrange_ind, -1.0, scores)
        scores = jnp.where(hrange_ind, -1.0, scores)
        dist_inds = (distsx > _DIST_T) | (distsy > _DIST_T) | (dists > 0.25)
        width_inds = br_xs < tl_xs
        height_inds = br_ys < tl_ys
        scores = jnp.where(cls_inds, -1.0, scores)
        scores = jnp.where(dist_inds, -1.0, scores)
        scores = jnp.where(width_inds, -1.0, scores)
        scores = jnp.where(height_inds, -1.0, scores)
        scores = scores.reshape(b, -1)
        scores, inds = lax.top_k(scores, min(_NUM_DETS, scores.shape[1]))
        scores = scores[:, :, None]
        bb = _gather_rows(bboxes.reshape(b, -1, 4), inds)
        clses = _gather_rows(tl_ce.reshape(b, -1, 1), inds).astype(jnp.float32)
        tl_sg = _gather_rows(tl_se.reshape(b, -1, 1), inds).astype(jnp.float32)
        br_sg = _gather_rows(br_se.reshape(b, -1, 1), inds).astype(jnp.float32)
        bb = bb * jnp.array([ws, hs, ws, hs], dtype=jnp.float32)
        dets.append(jnp.concatenate([bb, scores, tl_sg, br_sg, clses], axis=2))
    detections = jnp.concatenate(dets, axis=1)
    top_scores, top_inds = lax.top_k(detections[:, :, 4], 5 * _NUM_DETS)
    detections = _gather_rows(detections, top_inds)
    return detections


def kernel(tl_heat_0, br_heat_0, tl_regr_0, br_regr_0, tl_c_regr_0, br_c_regr_0,
           tl_heat_1, br_heat_1, tl_regr_1, br_regr_1, tl_c_regr_1, br_c_regr_1,
           tl_heat_2, br_heat_2, tl_regr_2, br_regr_2, tl_c_regr_2, br_c_regr_2,
           tl_heat_3, br_heat_3, tl_regr_3, br_regr_3, tl_c_regr_3, br_c_regr_3,
           tl_heat_4, br_heat_4, tl_regr_4, br_regr_4, tl_c_regr_4, br_c_regr_4):
    tl_heats = [tl_heat_0, tl_heat_1, tl_heat_2, tl_heat_3, tl_heat_4]
    br_heats = [br_heat_0, br_heat_1, br_heat_2, br_heat_3, br_heat_4]
    tl_regrs = [tl_regr_0, tl_regr_1, tl_regr_2, tl_regr_3, tl_regr_4]
    br_regrs = [br_regr_0, br_regr_1, br_regr_2, br_regr_3, br_regr_4]
    tl_c_regrs = [tl_c_regr_0, tl_c_regr_1, tl_c_regr_2, tl_c_regr_3, tl_c_regr_4]
    br_c_regrs = [br_c_regr_0, br_c_regr_1, br_c_regr_2, br_c_regr_3, br_c_regr_4]
    return _decode_all(tl_heats, br_heats, tl_regrs, br_regrs, tl_c_regrs, br_c_regrs)
